# baseline (speedup 1.0000x reference)
import functools

import jax
import jax.numpy as jnp
from jax.experimental import pallas as pl

MARGIN = 0.25
GAMMA = 256.0
O_N = -MARGIN
DELTA_N = MARGIN

BLOCK_B = 16


def _body(lab_ref, x_ref, o_ref):
    x = x_ref[...]
    cos = jnp.clip(x, -1.0, 1.0)
    alpha_n = jnp.maximum(cos - O_N, 0.0)
    logit_n = alpha_n * (cos - DELTA_N)
    col = jax.lax.broadcasted_iota(jnp.int32, x.shape, 1)
    is_label = col == lab_ref[...]
    o_ref[...] = jnp.where(is_label, cos, logit_n) * GAMMA


@functools.partial(jax.jit, static_argnums=())
def kernel(cos_theta, labels):
    b, c = cos_theta.shape
    lab2d = labels.astype(jnp.int32).reshape(b, 1)
    grid = (b // BLOCK_B,)
    return pl.pallas_call(
        _body,
        grid=grid,
        in_specs=[
            pl.BlockSpec((BLOCK_B, 1), lambda i: (i, 0)),
            pl.BlockSpec((BLOCK_B, c), lambda i: (i, 0)),
        ],
        out_specs=pl.BlockSpec((BLOCK_B, c), lambda i: (i, 0)),
        out_shape=jax.ShapeDtypeStruct((b, c), jnp.float32),
    )(lab2d, cos_theta)
